# baseline (speedup 1.0000x reference)
import jax
import jax.numpy as jnp
from jax.experimental import pallas as pl
from jax.experimental.pallas import tpu as pltpu


def _lstm_kernel(x_ref, wih_ref, whh_ref, b_ref, wout_ref, bout_ref, out_ref):
    seq, Bb, I = x_ref.shape
    H = whh_ref.shape[0]

    x2d = x_ref[...].reshape(seq * Bb, I).astype(jnp.bfloat16)
    gx = (
        jnp.dot(x2d, wih_ref[...], preferred_element_type=jnp.float32)
        + b_ref[...]
    )

    whh = whh_ref[...]

    h = jnp.zeros((Bb, H), jnp.float32)
    c = jnp.zeros((Bb, H), jnp.float32)
    for t in range(seq):
        gates = gx[t * Bb:(t + 1) * Bb, :] + jnp.dot(
            h, whh, preferred_element_type=jnp.float32)
        i_g = jax.nn.sigmoid(gates[:, 0 * H:1 * H])
        f_g = jax.nn.sigmoid(gates[:, 1 * H:2 * H])
        g_g = jnp.tanh(gates[:, 2 * H:3 * H])
        o_g = jax.nn.sigmoid(gates[:, 3 * H:4 * H])
        c = f_g * c + i_g * g_g
        h = o_g * jnp.tanh(c)

    out_ref[...] = (
        jnp.dot(h, wout_ref[...], preferred_element_type=jnp.float32)
        + bout_ref[...]
    ).astype(out_ref.dtype)


def kernel(x, w_ih, w_hh, b_ih, b_hh, w_out, b_out):
    seq, B, I = x.shape
    H = w_hh.shape[1]
    n_out = w_out.shape[0]
    n_out_pad = ((n_out + 127) // 128) * 128

    n_shards = 2
    Bb = B // n_shards

    x = x.astype(jnp.float32)
    wih_t = w_ih.T.astype(jnp.bfloat16)
    whh_t = w_hh.T.astype(jnp.float32)
    b = (b_ih + b_hh).reshape(1, 4 * H).astype(jnp.float32)
    wout_t = jnp.zeros((H, n_out_pad), jnp.float32).at[:, :n_out].set(w_out.T)
    bout = jnp.zeros((1, n_out_pad), jnp.float32).at[:, :n_out].set(
        b_out.reshape(1, n_out))

    grid_spec = pltpu.PrefetchScalarGridSpec(
        num_scalar_prefetch=0,
        grid=(n_shards,),
        in_specs=[
            pl.BlockSpec((seq, Bb, I), lambda i: (0, i, 0)),
            pl.BlockSpec((I, 4 * H), lambda i: (0, 0)),
            pl.BlockSpec((H, 4 * H), lambda i: (0, 0)),
            pl.BlockSpec((1, 4 * H), lambda i: (0, 0)),
            pl.BlockSpec((H, n_out_pad), lambda i: (0, 0)),
            pl.BlockSpec((1, n_out_pad), lambda i: (0, 0)),
        ],
        out_specs=pl.BlockSpec((Bb, n_out_pad), lambda i: (i, 0)),
    )

    out_pad = pl.pallas_call(
        _lstm_kernel,
        out_shape=jax.ShapeDtypeStruct((B, n_out_pad), jnp.float32),
        grid_spec=grid_spec,
        compiler_params=pltpu.CompilerParams(
            dimension_semantics=("parallel",)),
    )(x, wih_t, whh_t, b, wout_t, bout)

    return out_pad[:, :n_out].astype(x.dtype)
